# Initial kernel scaffold; baseline (speedup 1.0000x reference)
import jax
import jax.numpy as jnp
from jax.experimental import pallas as pl


def kernel(dec_x, dec_pc, enc_x, enc_pc, W1, b1, W2, b2):
    raise NotImplementedError("write your pallas kernel here")


# speedup vs baseline: 16.8321x; 16.8321x over previous
import functools

import jax
import jax.numpy as jnp
from jax import lax
from jax.experimental import pallas as pl
from jax.experimental.pallas import tpu as pltpu
from jax.experimental.pallas import tpu_sc as plsc

B = 8
N = 4096
M = 1024
K = 16
IN_DIM = 128
DIM = 256
TROW = 144
R2 = 128
R4 = 128

NC = 2
NS = 16
NW = NC * NS
CH = 128
NCHUNK = (B * M * K) // (NW * CH)


def _fps_body(px_ref, py_ref, pz_ref, fx_ref, fy_ref, fz_ref):
    px = px_ref[...]
    py = py_ref[...]
    pz = pz_ref[...]
    lanes_n = lax.broadcasted_iota(jnp.int32, (B, N), 1)
    lanes_m = lax.broadcasted_iota(jnp.int32, (B, M), 1)

    def body(i, st):
        dists, far, fx, fy, fz = st
        sel = (lanes_n == far).astype(jnp.float32)
        cx = jnp.sum(px * sel, axis=1, keepdims=True)
        cy = jnp.sum(py * sel, axis=1, keepdims=True)
        cz = jnp.sum(pz * sel, axis=1, keepdims=True)
        hit = lanes_m == i
        fx = jnp.where(hit, cx, fx)
        fy = jnp.where(hit, cy, fy)
        fz = jnp.where(hit, cz, fz)
        dx = px - cx
        dy = py - cy
        dz = pz - cz
        d = (dx * dx + dz * dz) + dy * dy
        dists = jnp.minimum(dists, d)
        mx = jnp.max(dists, axis=1, keepdims=True)
        far = jnp.min(jnp.where(dists == mx, lanes_n, N), axis=1, keepdims=True)
        return dists, far, fx, fy, fz

    init = (
        jnp.full((B, N), 1e10, jnp.float32),
        jnp.zeros((B, 1), jnp.int32),
        jnp.zeros((B, M), jnp.float32),
        jnp.zeros((B, M), jnp.float32),
        jnp.zeros((B, M), jnp.float32),
    )
    _, _, fx, fy, fz = lax.fori_loop(0, M, body, init)
    fx_ref[...] = fx
    fy_ref[...] = fy
    fz_ref[...] = fz


def _knn_body(pc1_ref, pc2t_ref, gidx_ref):
    bidx = pl.program_id(0)
    pc1 = pc1_ref[0]
    pc2t = pc2t_ref[0]
    d1 = jnp.sum(pc1 * pc1, axis=1, keepdims=True)
    d2 = jnp.sum(pc2t * pc2t, axis=0, keepdims=True)
    inner = jnp.dot(pc1, pc2t, preferred_element_type=jnp.float32)
    mat = (d1 + d2) - 2.0 * inner
    lanes = lax.broadcasted_iota(jnp.int32, (R2, N), 1)
    klanes = lax.broadcasted_iota(jnp.int32, (R2, K), 1)
    idx_acc = jnp.zeros((R2, K), jnp.int32)
    inf = jnp.float32(jnp.inf)
    for t in range(K):
        m = jnp.min(mat, axis=1, keepdims=True)
        idx = jnp.min(jnp.where(mat == m, lanes, N), axis=1, keepdims=True)
        idx_acc = jnp.where(klanes == t, idx, idx_acc)
        mat = jnp.where(lanes == idx, inf, mat)
    gidx_ref[0] = idx_acc + bidx * N


def _premix_body(x_ref, w_ref, o_ref):
    o_ref[...] = jnp.dot(x_ref[...], w_ref[...],
                         preferred_element_type=jnp.float32)


def _sc_gather_body(table_hbm, idx_hbm, out_hbm, idxv, rowsv, sem):
    c = lax.axis_index("c")
    s = lax.axis_index("s")
    wid = s * NC + c
    pltpu.sync_copy(idx_hbm.at[wid], idxv)

    def chunk(j, carry):
        pltpu.async_copy(table_hbm.at[idxv.at[j]], rowsv, sem).wait()
        pltpu.sync_copy(rowsv, out_hbm.at[pl.ds((wid * NCHUNK + j) * CH, CH)])
        return carry

    lax.fori_loop(0, NCHUNK, chunk, 0)


@functools.cache
def _sc_gather():
    return pl.kernel(
        _sc_gather_body,
        out_type=jax.ShapeDtypeStruct((B * M * K, DIM), jnp.float32),
        mesh=plsc.VectorSubcoreMesh(
            core_axis_name="c", subcore_axis_name="s", num_cores=NC),
        scratch_types=[
            pltpu.VMEM((NCHUNK, CH), jnp.int32),
            pltpu.VMEM((CH, DIM), jnp.float32),
            pltpu.SemaphoreType.DMA,
        ],
    )


def _mlp_body(g_ref, pc1_ref, w1pc_ref, b1_ref, w2_ref, b2_ref, out_ref):
    g = g_ref[...]
    corr = jnp.dot(pc1_ref[...], w1pc_ref[...],
                   preferred_element_type=jnp.float32)
    corr16 = jnp.broadcast_to(corr[:, None, :], (R4, K, DIM)).reshape(R4 * K, DIM)
    h = jnp.maximum(g + b1_ref[...] - corr16, 0.0)
    h = jnp.dot(h, w2_ref[...], preferred_element_type=jnp.float32)
    out_ref[...] = jnp.max(h.reshape(R4, K, DIM), axis=1) + b2_ref[...]


def kernel(dec_x, dec_pc, enc_x, enc_pc, W1, b1, W2, b2):
    f32 = jnp.float32
    pcT = jnp.transpose(dec_pc, (0, 2, 1))

    fx, fy, fz = pl.pallas_call(
        _fps_body,
        out_shape=[jax.ShapeDtypeStruct((B, M), f32)] * 3,
    )(pcT[:, 0], pcT[:, 1], pcT[:, 2])
    pc_fps = jnp.stack([fx, fy, fz], axis=-1)

    pc1p = jnp.concatenate([pc_fps, jnp.zeros((B, M, 5), f32)], axis=-1)
    pc2t = jnp.concatenate([pcT, jnp.zeros((B, 5, N), f32)], axis=1)
    gidx = pl.pallas_call(
        _knn_body,
        grid=(B, M // R2),
        in_specs=[
            pl.BlockSpec((1, R2, 8), lambda b, t: (b, t, 0)),
            pl.BlockSpec((1, 8, N), lambda b, t: (b, 0, 0)),
        ],
        out_specs=pl.BlockSpec((1, R2, K), lambda b, t: (b, t, 0)),
        out_shape=jax.ShapeDtypeStruct((B, M, K), jnp.int32),
    )(pc1p, pc2t)

    X = jnp.concatenate(
        [dec_x, dec_pc, jnp.zeros((B, N, TROW - IN_DIM - 3), f32)], axis=-1
    ).reshape(B * N, TROW)
    W1pad = jnp.concatenate(
        [W1[3:], W1[:3], jnp.zeros((TROW - IN_DIM - 3, DIM), f32)], axis=0)
    table = pl.pallas_call(
        _premix_body,
        grid=(B * N // 1024,),
        in_specs=[
            pl.BlockSpec((1024, TROW), lambda t: (t, 0)),
            pl.BlockSpec((TROW, DIM), lambda t: (0, 0)),
        ],
        out_specs=pl.BlockSpec((1024, DIM), lambda t: (t, 0)),
        out_shape=jax.ShapeDtypeStruct((B * N, DIM), f32),
    )(X, W1pad)
    g = _sc_gather()(table, gidx.reshape(NW, NCHUNK, CH))

    W1pcp = jnp.concatenate([W1[:3], jnp.zeros((5, DIM), f32)], axis=0)
    out = pl.pallas_call(
        _mlp_body,
        grid=(B * M // R4,),
        in_specs=[
            pl.BlockSpec((R4 * K, DIM), lambda t: (t, 0)),
            pl.BlockSpec((R4, 8), lambda t: (t, 0)),
            pl.BlockSpec((8, DIM), lambda t: (0, 0)),
            pl.BlockSpec((1, DIM), lambda t: (0, 0)),
            pl.BlockSpec((DIM, DIM), lambda t: (0, 0)),
            pl.BlockSpec((1, DIM), lambda t: (0, 0)),
        ],
        out_specs=pl.BlockSpec((R4, DIM), lambda t: (t, 0)),
        out_shape=jax.ShapeDtypeStruct((B * M, DIM), f32),
    )(g, pc1p.reshape(B * M, 8), W1pcp,
      b1.reshape(1, DIM), W2, b2.reshape(1, DIM))

    return (out.reshape(B, M, DIM), pc_fps, enc_x, enc_pc)
